# baseline (speedup 1.0000x reference)
import functools

import jax
import jax.numpy as jnp
from jax import lax
from jax.experimental import pallas as pl
from jax.experimental.pallas import tpu as pltpu
from jax.experimental.pallas import tpu_sc as plsc

N = 1048576
LANES = 16
B = 512
NG = B // LANES
IDX_CHUNK = 128
NSEG = 2 * B // IDX_CHUNK

_DEGS = ((0, 3, 2048), (1, 9, 1024), (2, 15, 512))

_COLMAP = []
for _c in range(3):
    _COLMAP.append((0, _c, _c * 9 + 0))
    for _k in range(3):
        _COLMAP.append((1, 3 * _c + _k, _c * 9 + 1 + _k))
    for _k in range(5):
        _COLMAP.append((2, 5 * _c + _k, _c * 9 + 4 + _k))


def kernel(uv_coords, tex0, tex1, tex2):
    tabs = [tex0.reshape(-1, 16), tex1.reshape(-1, 16), tex2.reshape(-1, 16)]
    maxrow = [t.shape[0] - 1 for t in tabs]

    info = plsc.get_sparse_core_info()
    nc, ns = info.num_cores, info.num_subcores
    nw = nc * ns
    pts_per_tile = N // nw
    n_chunks = pts_per_tile // B

    @functools.partial(
        pl.kernel,
        out_type=jax.ShapeDtypeStruct((N, 3, 9), jnp.float32),
        mesh=plsc.VectorSubcoreMesh(core_axis_name="c", subcore_axis_name="s"),
        compiler_params=pltpu.CompilerParams(
            needs_layout_passes=False, use_tc_tiling_on_sc=False),
        scratch_types=[
            pltpu.VMEM((B, 2), jnp.float32),
            pltpu.VMEM((2 * B,), jnp.int32),
            pltpu.VMEM((2 * B,), jnp.int32),
            pltpu.VMEM((2 * B,), jnp.int32),
            pltpu.VMEM((B,), jnp.int32),
            pltpu.VMEM((B,), jnp.int32),
            pltpu.VMEM((B,), jnp.int32),
            pltpu.VMEM((2 * B, 16), jnp.float32),
            pltpu.VMEM((2 * B, 16), jnp.float32),
            pltpu.VMEM((2 * B, 16), jnp.float32),
            pltpu.VMEM((B, 3, 9), jnp.float32),
            pltpu.SemaphoreType.DMA,
        ],
    )
    def sc_kernel(uv_hbm, t0_hbm, t1_hbm, t2_hbm, out_hbm,
                  uv_v, i0_v, i1_v, i2_v, bo0_v, bo1_v, bo2_v,
                  g0_v, g1_v, g2_v, out_v, sem):
        wid = lax.axis_index("s") * nc + lax.axis_index("c")
        iota = lax.iota(jnp.int32, LANES)
        zeros = jnp.zeros((LANES,), jnp.int32)
        idx_refs = (i0_v, i1_v, i2_v)
        bo_refs = (bo0_v, bo1_v, bo2_v)
        g_refs = (g0_v, g1_v, g2_v)

        def chunk_body(ci, _):
            base = wid * pts_per_tile + ci * B
            pltpu.sync_copy(uv_hbm.at[pl.ds(base, B)], uv_v)

            def gen_body(g, _):
                p = iota + g * LANES
                p2 = p << 1
                u = plsc.load_gather(uv_v, [p, zeros])
                v = plsc.load_gather(uv_v, [p, zeros + 1])
                ix = jnp.clip((u * 2048.0).astype(jnp.int32), 0, 2047)
                iy = jnp.clip((v * 2048.0).astype(jnp.int32), 0, 2047)
                for s, d, w in _DEGS:
                    sh = s
                    texel = ((iy >> sh) << (11 - sh)) + (ix >> sh)
                    o = texel * d
                    r = o >> 4
                    r2 = jnp.minimum(r + 1, maxrow[s])
                    plsc.store_scatter(idx_refs[s], [p2], r)
                    plsc.store_scatter(idx_refs[s], [p2 + 1], r2)
                    plsc.store_scatter(bo_refs[s], [p], (p << 5) + (o & 15))
                return _

            lax.fori_loop(0, NG, gen_body, None)

            copies = []
            for s in range(3):
                tab = (t0_hbm, t1_hbm, t2_hbm)[s]
                for i in range(NSEG):
                    sl = pl.ds(i * IDX_CHUNK, IDX_CHUNK)
                    copies.append(pltpu.async_copy(
                        tab.at[idx_refs[s].at[sl]],
                        g_refs[s].at[sl], sem))
            for c in copies:
                c.wait()

            def shuf_body(g, _):
                p = iota + g * LANES
                bo = [plsc.load_gather(bo_refs[s], [p]) for s in range(3)]
                for s, col, j in _COLMAP:
                    w = bo[s] + col
                    val = plsc.load_gather(g_refs[s], [w >> 4, w & 15])
                    plsc.store_scatter(
                        out_v, [p, zeros + (j // 9), zeros + (j % 9)], val)
                return _

            lax.fori_loop(0, NG, shuf_body, None)
            pltpu.sync_copy(out_v, out_hbm.at[pl.ds(base, B)])
            return _

        lax.fori_loop(0, n_chunks, chunk_body, None)

    return sc_kernel(uv_coords, *tabs)


# speedup vs baseline: 1.2599x; 1.2599x over previous
import functools

import jax
import jax.numpy as jnp
from jax import lax
from jax.experimental import pallas as pl
from jax.experimental.pallas import tpu as pltpu
from jax.experimental.pallas import tpu_sc as plsc

N = 1048576
NBLK = N // 128
LANES = 16
B = 256
BBLK = B // 128
NG = B // LANES
IDX_CHUNK = 128
NSEG = 2 * B // IDX_CHUNK

_DEGS = ((0, 3), (1, 9), (2, 15))

_COLMAP = []
for _c in range(3):
    _COLMAP.append((0, _c, _c * 9 + 0))
    for _k in range(3):
        _COLMAP.append((1, 3 * _c + _k, _c * 9 + 1 + _k))
    for _k in range(5):
        _COLMAP.append((2, 5 * _c + _k, _c * 9 + 4 + _k))


def kernel(uv_coords, tex0, tex1, tex2):
    tabs = [tex0.reshape(-1, 16), tex1.reshape(-1, 16), tex2.reshape(-1, 16)]
    maxrow = [t.shape[0] - 1 for t in tabs]
    uv_blk = uv_coords.reshape(NBLK, 128, 2).transpose(0, 2, 1)

    info = plsc.get_sparse_core_info()
    nc, ns = info.num_cores, info.num_subcores
    nw = nc * ns
    pts_per_tile = N // nw
    n_chunks = pts_per_tile // B

    @functools.partial(
        pl.kernel,
        out_type=jax.ShapeDtypeStruct((9, NBLK, 4, 128), jnp.float32),
        mesh=plsc.VectorSubcoreMesh(core_axis_name="c", subcore_axis_name="s"),
        compiler_params=pltpu.CompilerParams(
            needs_layout_passes=False, use_tc_tiling_on_sc=False),
        scratch_types=[
            pltpu.VMEM((BBLK, 2, 128), jnp.float32),
            pltpu.VMEM((2 * B,), jnp.int32),
            pltpu.VMEM((2 * B,), jnp.int32),
            pltpu.VMEM((2 * B,), jnp.int32),
            pltpu.VMEM((B,), jnp.int32),
            pltpu.VMEM((B,), jnp.int32),
            pltpu.VMEM((B,), jnp.int32),
            pltpu.VMEM((2 * B, 16), jnp.float32),
            pltpu.VMEM((2 * B, 16), jnp.float32),
            pltpu.VMEM((2 * B, 16), jnp.float32),
            pltpu.VMEM((9, BBLK, 4, 128), jnp.float32),
            pltpu.SemaphoreType.DMA,
        ],
    )
    def sc_kernel(uv_hbm, t0_hbm, t1_hbm, t2_hbm, out_hbm,
                  uv_v, i0_v, i1_v, i2_v, bo0_v, bo1_v, bo2_v,
                  g0_v, g1_v, g2_v, out_v, sem):
        wid = lax.axis_index("s") * nc + lax.axis_index("c")
        iota = lax.iota(jnp.int32, LANES)
        zeros = jnp.zeros((LANES,), jnp.int32)
        idx_refs = (i0_v, i1_v, i2_v)
        bo_refs = (bo0_v, bo1_v, bo2_v)
        g_refs = (g0_v, g1_v, g2_v)

        def chunk_body(ci, _):
            base = wid * pts_per_tile + ci * B
            blk0 = base // 128
            pltpu.sync_copy(uv_hbm.at[pl.ds(blk0, BBLK)], uv_v)

            def gen_body(g, _):
                q = iota + g * LANES
                qb = q >> 7
                ql = q & 127
                q2 = q << 1
                u = plsc.load_gather(uv_v, [qb, zeros, ql])
                v = plsc.load_gather(uv_v, [qb, zeros + 1, ql])
                ix = jnp.clip((u * 2048.0).astype(jnp.int32), 0, 2047)
                iy = jnp.clip((v * 2048.0).astype(jnp.int32), 0, 2047)
                for s, d in _DEGS:
                    texel = ((iy >> s) << (11 - s)) + (ix >> s)
                    o = texel * d
                    r = o >> 4
                    r2 = jnp.minimum(r + 1, maxrow[s])
                    plsc.store_scatter(idx_refs[s], [q2], r)
                    plsc.store_scatter(idx_refs[s], [q2 + 1], r2)
                    plsc.store_scatter(bo_refs[s], [q], (q << 5) + (o & 15))
                return _

            lax.fori_loop(0, NG, gen_body, None)

            copies = []
            for s in range(3):
                tab = (t0_hbm, t1_hbm, t2_hbm)[s]
                for i in range(NSEG):
                    sl = pl.ds(i * IDX_CHUNK, IDX_CHUNK)
                    copies.append(pltpu.async_copy(
                        tab.at[idx_refs[s].at[sl]], g_refs[s].at[sl], sem))
            for c in copies:
                c.wait()

            def shuf_body(g, _):
                q = iota + g * LANES
                qb = q >> 7
                ql = q & 127
                bo = [plsc.load_gather(bo_refs[s], [q]) for s in range(3)]
                for s, col, j in _COLMAP:
                    w = bo[s] + col
                    val = plsc.load_gather(g_refs[s], [w >> 4, w & 15])
                    plsc.store_scatter(
                        out_v,
                        [zeros + (j % 9), qb, zeros + (j // 9), ql], val)
                return _

            lax.fori_loop(0, NG, shuf_body, None)
            for k in range(9):
                pltpu.sync_copy(out_v.at[k],
                                out_hbm.at[k, pl.ds(blk0, BBLK)])
            return _

        lax.fori_loop(0, n_chunks, chunk_body, None)

    out = sc_kernel(uv_blk, *tabs)
    return (out[:, :, :3, :]
            .transpose(1, 3, 2, 0)
            .reshape(N, 3, 9))


# speedup vs baseline: 12.4035x; 9.8445x over previous
import functools

import jax
import jax.numpy as jnp
from jax import lax
from jax.experimental import pallas as pl
from jax.experimental.pallas import tpu as pltpu
from jax.experimental.pallas import tpu_sc as plsc

N = 1048576
NBLK = N // 128
LANES = 16
B = 256
BBLK = B // 128
NG = B // LANES
IDX_CHUNK = 128

PLANE_ROWS = 2048 * 2048 // 8

_COLMAP = []
for _c in range(3):
    _COLMAP.append((0, _c, _c * 9 + 0))
    for _k in range(3):
        _COLMAP.append((1, 3 * _c + _k, _c * 9 + 1 + _k))
    for _k in range(5):
        _COLMAP.append((2, 5 * _c + _k, _c * 9 + 4 + _k))


def kernel(uv_coords, tex0, tex1, tex2):
    t0_rows = (tex0.transpose(2, 0, 1)
               .reshape(3, 256, 8, 16, 128)
               .transpose(0, 1, 3, 2, 4)
               .reshape(3 * PLANE_ROWS, 8))
    tabs = [tex1.reshape(-1, 16), tex2.reshape(-1, 16)]
    maxrow = [t.shape[0] - 1 for t in tabs]
    uv_blk = uv_coords.reshape(NBLK, 128, 2).transpose(0, 2, 1)

    info = plsc.get_sparse_core_info()
    nc, ns = info.num_cores, info.num_subcores
    nw = nc * ns
    pts_per_tile = N // nw
    n_chunks = pts_per_tile // B

    @functools.partial(
        pl.kernel,
        out_type=jax.ShapeDtypeStruct((9, NBLK, 4, 128), jnp.float32),
        mesh=plsc.VectorSubcoreMesh(core_axis_name="c", subcore_axis_name="s"),
        compiler_params=pltpu.CompilerParams(
            needs_layout_passes=False, use_tc_tiling_on_sc=False),
        scratch_types=[
            pltpu.VMEM((BBLK, 2, 128), jnp.float32),
            pltpu.VMEM((3 * B,), jnp.int32),
            pltpu.VMEM((2 * B,), jnp.int32),
            pltpu.VMEM((2 * B,), jnp.int32),
            pltpu.VMEM((B,), jnp.int32),
            pltpu.VMEM((B,), jnp.int32),
            pltpu.VMEM((B,), jnp.int32),
            pltpu.VMEM((3 * B, 8), jnp.float32),
            pltpu.VMEM((2 * B, 16), jnp.float32),
            pltpu.VMEM((2 * B, 16), jnp.float32),
            pltpu.VMEM((9, BBLK, 4, 128), jnp.float32),
            pltpu.SemaphoreType.DMA,
        ],
    )
    def sc_kernel(uv_hbm, t0_hbm, t1_hbm, t2_hbm, out_hbm,
                  uv_v, i0_v, i1_v, i2_v, bo0_v, bo1_v, bo2_v,
                  g0_v, g1_v, g2_v, out_v, sem):
        wid = lax.axis_index("s") * nc + lax.axis_index("c")
        iota = lax.iota(jnp.int32, LANES)
        zeros = jnp.zeros((LANES,), jnp.int32)

        def chunk_body(ci, _):
            base = wid * pts_per_tile + ci * B
            blk0 = base // 128
            pltpu.sync_copy(uv_hbm.at[pl.ds(blk0, BBLK)], uv_v)

            def gen_body(g, _):
                q = iota + g * LANES
                qb = q >> 7
                ql = q & 127
                q2 = q << 1
                q3 = q2 + q
                u = plsc.load_gather(uv_v, [qb, zeros, ql])
                v = plsc.load_gather(uv_v, [qb, zeros + 1, ql])
                ix = jnp.clip((u * 2048.0).astype(jnp.int32), 0, 2047)
                iy = jnp.clip((v * 2048.0).astype(jnp.int32), 0, 2047)
                w = ((((iy >> 3) << 4) + (ix >> 7)) << 10) \
                    + ((iy & 7) << 7) + (ix & 127)
                r0 = w >> 3
                plsc.store_scatter(i0_v, [q3], r0)
                plsc.store_scatter(i0_v, [q3 + 1], r0 + PLANE_ROWS)
                plsc.store_scatter(i0_v, [q3 + 2], r0 + 2 * PLANE_ROWS)
                plsc.store_scatter(bo0_v, [q], ix & 7)
                for s, d in ((0, 9), (1, 15)):
                    sh = s + 1
                    texel = ((iy >> sh) << (11 - sh)) + (ix >> sh)
                    o = texel * d
                    r = o >> 4
                    r2 = jnp.minimum(r + 1, maxrow[s])
                    iref = (i1_v, i2_v)[s]
                    plsc.store_scatter(iref, [q2], r)
                    plsc.store_scatter(iref, [q2 + 1], r2)
                    plsc.store_scatter((bo1_v, bo2_v)[s], [q],
                                       (q << 5) + (o & 15))
                return _

            lax.fori_loop(0, NG, gen_body, None)

            copies = []
            for i in range(3 * B // IDX_CHUNK):
                sl = pl.ds(i * IDX_CHUNK, IDX_CHUNK)
                copies.append(pltpu.async_copy(
                    t0_hbm.at[i0_v.at[sl]], g0_v.at[sl], sem))
            for s in range(2):
                tab = (t1_hbm, t2_hbm)[s]
                iref = (i1_v, i2_v)[s]
                gref = (g1_v, g2_v)[s]
                for i in range(2 * B // IDX_CHUNK):
                    sl = pl.ds(i * IDX_CHUNK, IDX_CHUNK)
                    copies.append(pltpu.async_copy(
                        tab.at[iref.at[sl]], gref.at[sl], sem))
            for c in copies:
                c.wait()

            def shuf_body(g, _):
                q = iota + g * LANES
                qb = q >> 7
                ql = q & 127
                q3 = (q << 1) + q
                x7 = plsc.load_gather(bo0_v, [q])
                bo1 = plsc.load_gather(bo1_v, [q])
                bo2 = plsc.load_gather(bo2_v, [q])
                for s, col, j in _COLMAP:
                    if s == 0:
                        val = plsc.load_gather(g0_v, [q3 + col, x7])
                    else:
                        w = (bo1, bo2)[s - 1] + col
                        val = plsc.load_gather(
                            (g1_v, g2_v)[s - 1], [w >> 4, w & 15])
                    plsc.store_scatter(
                        out_v,
                        [zeros + (j % 9), qb, zeros + (j // 9), ql], val)
                return _

            lax.fori_loop(0, NG, shuf_body, None)
            for k in range(9):
                pltpu.sync_copy(out_v.at[k],
                                out_hbm.at[k, pl.ds(blk0, BBLK)])
            return _

        lax.fori_loop(0, n_chunks, chunk_body, None)

    out = sc_kernel(uv_blk, t0_rows, *tabs)
    return (out[:, :, :3, :]
            .transpose(1, 3, 2, 0)
            .reshape(N, 3, 9))


# speedup vs baseline: 17.4321x; 1.4054x over previous
import functools

import jax
import jax.numpy as jnp
from jax import lax
from jax.experimental import pallas as pl
from jax.experimental.pallas import tpu as pltpu
from jax.experimental.pallas import tpu_sc as plsc

N = 1048576
NBLK = N // 128
LANES = 16
B = 256
BBLK = B // 128
NG = B // LANES
IDX_CHUNK = 128

PLANE_ROWS = 2048 * 2048 // 8

_COLMAP = []
for _c in range(3):
    _COLMAP.append((0, _c, _c * 9 + 0))
    for _k in range(3):
        _COLMAP.append((1, 3 * _c + _k, _c * 9 + 1 + _k))
    for _k in range(5):
        _COLMAP.append((2, 5 * _c + _k, _c * 9 + 4 + _k))


def _scratch_set():
    return [
        pltpu.VMEM((BBLK, 2, 128), jnp.float32),
        pltpu.VMEM((3 * B,), jnp.int32),
        pltpu.VMEM((2 * B,), jnp.int32),
        pltpu.VMEM((2 * B,), jnp.int32),
        pltpu.VMEM((B,), jnp.int32),
        pltpu.VMEM((B,), jnp.int32),
        pltpu.VMEM((B,), jnp.int32),
        pltpu.VMEM((3 * B, 8), jnp.float32),
        pltpu.VMEM((2 * B, 16), jnp.float32),
        pltpu.VMEM((2 * B, 16), jnp.float32),
        pltpu.VMEM((9, BBLK, 4, 128), jnp.float32),
        pltpu.SemaphoreType.DMA,
        pltpu.SemaphoreType.DMA,
    ]


def kernel(uv_coords, tex0, tex1, tex2):
    t0_rows = (tex0.transpose(2, 0, 1)
               .reshape(3, 256, 8, 16, 128)
               .transpose(0, 1, 3, 2, 4)
               .reshape(3 * PLANE_ROWS, 8))
    tabs = [tex1.reshape(-1, 16), tex2.reshape(-1, 16)]
    maxrow = [t.shape[0] - 1 for t in tabs]
    uv_blk = uv_coords.reshape(NBLK, 128, 2).transpose(0, 2, 1)

    info = plsc.get_sparse_core_info()
    nc, ns = info.num_cores, info.num_subcores
    nw = nc * ns
    pts_per_tile = N // nw
    n_chunks = pts_per_tile // B
    n_pairs = n_chunks // 2

    @functools.partial(
        pl.kernel,
        out_type=jax.ShapeDtypeStruct((9, NBLK, 4, 128), jnp.float32),
        mesh=plsc.VectorSubcoreMesh(core_axis_name="c", subcore_axis_name="s"),
        compiler_params=pltpu.CompilerParams(
            needs_layout_passes=False, use_tc_tiling_on_sc=False),
        scratch_types=_scratch_set() + _scratch_set(),
    )
    def sc_kernel(uv_hbm, t0_hbm, t1_hbm, t2_hbm, out_hbm, *scr):
        bufA, bufB = scr[:13], scr[13:]
        wid = lax.axis_index("s") * nc + lax.axis_index("c")
        iota = lax.iota(jnp.int32, LANES)
        zeros = jnp.zeros((LANES,), jnp.int32)

        def chunk_base(ci):
            return wid * pts_per_tile + ci * B

        def fire(ci, buf):
            (uv_v, i0_v, i1_v, i2_v, bo0_v, bo1_v, bo2_v,
             g0_v, g1_v, g2_v, out_v, sem_g, sem_o) = buf
            blk0 = chunk_base(ci) // 128
            pltpu.sync_copy(uv_hbm.at[pl.ds(blk0, BBLK)], uv_v)

            def gen_body(g, _):
                q = iota + g * LANES
                qb = q >> 7
                ql = q & 127
                q2 = q << 1
                q3 = q2 + q
                u = plsc.load_gather(uv_v, [qb, zeros, ql])
                v = plsc.load_gather(uv_v, [qb, zeros + 1, ql])
                ix = jnp.clip((u * 2048.0).astype(jnp.int32), 0, 2047)
                iy = jnp.clip((v * 2048.0).astype(jnp.int32), 0, 2047)
                w = ((((iy >> 3) << 4) + (ix >> 7)) << 10) \
                    + ((iy & 7) << 7) + (ix & 127)
                r0 = w >> 3
                plsc.store_scatter(i0_v, [q3], r0)
                plsc.store_scatter(i0_v, [q3 + 1], r0 + PLANE_ROWS)
                plsc.store_scatter(i0_v, [q3 + 2], r0 + 2 * PLANE_ROWS)
                plsc.store_scatter(bo0_v, [q], ix & 7)
                for s, d in ((0, 9), (1, 15)):
                    sh = s + 1
                    texel = ((iy >> sh) << (11 - sh)) + (ix >> sh)
                    o = texel * d
                    r = o >> 4
                    r2 = jnp.minimum(r + 1, maxrow[s])
                    iref = (i1_v, i2_v)[s]
                    plsc.store_scatter(iref, [q2], r)
                    plsc.store_scatter(iref, [q2 + 1], r2)
                    plsc.store_scatter((bo1_v, bo2_v)[s], [q],
                                       (q << 5) + (o & 15))
                return _

            lax.fori_loop(0, NG, gen_body, None)
            for i in range(3 * B // IDX_CHUNK):
                sl = pl.ds(i * IDX_CHUNK, IDX_CHUNK)
                pltpu.async_copy(t0_hbm.at[i0_v.at[sl]], g0_v.at[sl], sem_g)
            for s in range(2):
                tab = (t1_hbm, t2_hbm)[s]
                iref = (i1_v, i2_v)[s]
                gref = (g1_v, g2_v)[s]
                for i in range(2 * B // IDX_CHUNK):
                    sl = pl.ds(i * IDX_CHUNK, IDX_CHUNK)
                    pltpu.async_copy(tab.at[iref.at[sl]], gref.at[sl], sem_g)

        def drain_gathers(buf):
            (uv_v, i0_v, i1_v, i2_v, bo0_v, bo1_v, bo2_v,
             g0_v, g1_v, g2_v, out_v, sem_g, sem_o) = buf
            for i in range(3 * B // IDX_CHUNK):
                sl = pl.ds(i * IDX_CHUNK, IDX_CHUNK)
                pltpu.make_async_copy(
                    t0_hbm.at[i0_v.at[sl]], g0_v.at[sl], sem_g).wait()
            for s in range(2):
                tab = (t1_hbm, t2_hbm)[s]
                iref = (i1_v, i2_v)[s]
                gref = (g1_v, g2_v)[s]
                for i in range(2 * B // IDX_CHUNK):
                    sl = pl.ds(i * IDX_CHUNK, IDX_CHUNK)
                    pltpu.make_async_copy(
                        tab.at[iref.at[sl]], gref.at[sl], sem_g).wait()

        def drain_out(ci, buf):
            out_v, sem_o = buf[10], buf[12]
            blk0 = chunk_base(ci) // 128
            pltpu.make_async_copy(
                out_v, out_hbm.at[:, pl.ds(blk0, BBLK)], sem_o).wait()

        def shuffle_and_out(ci, buf):
            (uv_v, i0_v, i1_v, i2_v, bo0_v, bo1_v, bo2_v,
             g0_v, g1_v, g2_v, out_v, sem_g, sem_o) = buf
            blk0 = chunk_base(ci) // 128

            def shuf_body(g, _):
                q = iota + g * LANES
                qb = q >> 7
                ql = q & 127
                q3 = (q << 1) + q
                x7 = plsc.load_gather(bo0_v, [q])
                bo1 = plsc.load_gather(bo1_v, [q])
                bo2 = plsc.load_gather(bo2_v, [q])
                for s, col, j in _COLMAP:
                    if s == 0:
                        val = plsc.load_gather(g0_v, [q3 + col, x7])
                    else:
                        w = (bo1, bo2)[s - 1] + col
                        val = plsc.load_gather(
                            (g1_v, g2_v)[s - 1], [w >> 4, w & 15])
                    plsc.store_scatter(
                        out_v,
                        [zeros + (j % 9), qb, zeros + (j // 9), ql], val)
                return _

            lax.fori_loop(0, NG, shuf_body, None)
            pltpu.async_copy(out_v, out_hbm.at[:, pl.ds(blk0, BBLK)], sem_o)

        fire(0, bufA)

        def pair_body(cp, _):
            ca = cp * 2
            cb = ca + 1
            fire(cb, bufB)
            drain_gathers(bufA)

            @pl.when(cp > 0)
            def _older_a():
                drain_out(ca, bufA)

            shuffle_and_out(ca, bufA)

            @pl.when(cp < n_pairs - 1)
            def _next_a():
                fire(ca + 2, bufA)

            drain_gathers(bufB)

            @pl.when(cp > 0)
            def _older_b():
                drain_out(cb, bufB)

            shuffle_and_out(cb, bufB)
            return _

        lax.fori_loop(0, n_pairs, pair_body, None)
        drain_out(0, bufA)
        drain_out(0, bufB)

    out = sc_kernel(uv_blk, t0_rows, *tabs)
    return (out[:, :, :3, :]
            .transpose(1, 3, 2, 0)
            .reshape(N, 3, 9))


# speedup vs baseline: 18.8569x; 1.0817x over previous
import functools

import jax
import jax.numpy as jnp
from jax import lax
from jax.experimental import pallas as pl
from jax.experimental.pallas import tpu as pltpu
from jax.experimental.pallas import tpu_sc as plsc

N = 1048576
NBLK = N // 128
LANES = 16
B = 256
BBLK = B // 128
NG = B // LANES
IDX_CHUNK = 128

PLANE_ROWS = 2048 * 2048 // 8

_COLMAP = []
for _c in range(3):
    _COLMAP.append((0, _c, _c * 9 + 0))
    for _k in range(3):
        _COLMAP.append((1, 3 * _c + _k, _c * 9 + 1 + _k))
    for _k in range(5):
        _COLMAP.append((2, 5 * _c + _k, _c * 9 + 4 + _k))


def _scratch_set():
    return [
        pltpu.VMEM((BBLK, 2, 128), jnp.float32),
        pltpu.VMEM((3 * B,), jnp.int32),
        pltpu.VMEM((2 * B,), jnp.int32),
        pltpu.VMEM((2 * B,), jnp.int32),
        pltpu.VMEM((B,), jnp.int32),
        pltpu.VMEM((B,), jnp.int32),
        pltpu.VMEM((B,), jnp.int32),
        pltpu.VMEM((3 * B, 8), jnp.float32),
        pltpu.VMEM((2 * B, 16), jnp.float32),
        pltpu.VMEM((2 * B, 16), jnp.float32),
        pltpu.VMEM((9, BBLK, 4, 128), jnp.float32),
        pltpu.SemaphoreType.DMA,
        pltpu.SemaphoreType.DMA,
    ]


def kernel(uv_coords, tex0, tex1, tex2):
    t0_rows = (tex0.transpose(2, 0, 1)
               .reshape(3, 256, 8, 16, 128)
               .transpose(0, 1, 3, 2, 4)
               .reshape(3 * PLANE_ROWS, 8))
    tabs = [tex1.reshape(-1, 16), tex2.reshape(-1, 16)]
    maxrow = [t.shape[0] - 1 for t in tabs]
    uv_blk = uv_coords.reshape(NBLK, 128, 2).transpose(0, 2, 1)

    info = plsc.get_sparse_core_info()
    nc, ns = info.num_cores, info.num_subcores
    nw = nc * ns
    pts_per_tile = N // nw
    n_chunks = pts_per_tile // B
    n_pairs = n_chunks // 2

    @functools.partial(
        pl.kernel,
        out_type=jax.ShapeDtypeStruct((9, NBLK, 4, 128), jnp.float32),
        mesh=plsc.VectorSubcoreMesh(core_axis_name="c", subcore_axis_name="s"),
        compiler_params=pltpu.CompilerParams(
            needs_layout_passes=False, use_tc_tiling_on_sc=False),
        scratch_types=_scratch_set() + _scratch_set(),
    )
    def sc_kernel(uv_hbm, t0_hbm, t1_hbm, t2_hbm, out_hbm, *scr):
        bufA, bufB = scr[:13], scr[13:]
        wid = lax.axis_index("s") * nc + lax.axis_index("c")
        iota = lax.iota(jnp.int32, LANES)
        zeros = jnp.zeros((LANES,), jnp.int32)

        def chunk_base(ci):
            return wid * pts_per_tile + ci * B

        def fire(ci, buf):
            (uv_v, i0_v, i1_v, i2_v, bo0_v, bo1_v, bo2_v,
             g0_v, g1_v, g2_v, out_v, sem_g, sem_o) = buf
            blk0 = chunk_base(ci) // 128
            pltpu.sync_copy(uv_hbm.at[pl.ds(blk0, BBLK)], uv_v)

            @plsc.parallel_loop(0, NG, 1, unroll=2)
            def gen_body(g):
                q = iota + g * LANES
                qb = q >> 7
                ql = q & 127
                q2 = q << 1
                q3 = q2 + q
                u = plsc.load_gather(uv_v, [qb, zeros, ql])
                v = plsc.load_gather(uv_v, [qb, zeros + 1, ql])
                ix = jnp.clip((u * 2048.0).astype(jnp.int32), 0, 2047)
                iy = jnp.clip((v * 2048.0).astype(jnp.int32), 0, 2047)
                w = ((((iy >> 3) << 4) + (ix >> 7)) << 10) \
                    + ((iy & 7) << 7) + (ix & 127)
                r0 = w >> 3
                plsc.store_scatter(i0_v, [q3], r0)
                plsc.store_scatter(i0_v, [q3 + 1], r0 + PLANE_ROWS)
                plsc.store_scatter(i0_v, [q3 + 2], r0 + 2 * PLANE_ROWS)
                plsc.store_scatter(bo0_v, [q], ix & 7)
                for s, d in ((0, 9), (1, 15)):
                    sh = s + 1
                    texel = ((iy >> sh) << (11 - sh)) + (ix >> sh)
                    o = texel * d
                    r = o >> 4
                    r2 = jnp.minimum(r + 1, maxrow[s])
                    iref = (i1_v, i2_v)[s]
                    plsc.store_scatter(iref, [q2], r)
                    plsc.store_scatter(iref, [q2 + 1], r2)
                    plsc.store_scatter((bo1_v, bo2_v)[s], [q],
                                       (q << 5) + (o & 15))

            for i in range(3 * B // IDX_CHUNK):
                sl = pl.ds(i * IDX_CHUNK, IDX_CHUNK)
                pltpu.async_copy(t0_hbm.at[i0_v.at[sl]], g0_v.at[sl], sem_g)
            for s in range(2):
                tab = (t1_hbm, t2_hbm)[s]
                iref = (i1_v, i2_v)[s]
                gref = (g1_v, g2_v)[s]
                for i in range(2 * B // IDX_CHUNK):
                    sl = pl.ds(i * IDX_CHUNK, IDX_CHUNK)
                    pltpu.async_copy(tab.at[iref.at[sl]], gref.at[sl], sem_g)

        def drain_gathers(buf):
            (uv_v, i0_v, i1_v, i2_v, bo0_v, bo1_v, bo2_v,
             g0_v, g1_v, g2_v, out_v, sem_g, sem_o) = buf
            for i in range(3 * B // IDX_CHUNK):
                sl = pl.ds(i * IDX_CHUNK, IDX_CHUNK)
                pltpu.make_async_copy(
                    t0_hbm.at[i0_v.at[sl]], g0_v.at[sl], sem_g).wait()
            for s in range(2):
                tab = (t1_hbm, t2_hbm)[s]
                iref = (i1_v, i2_v)[s]
                gref = (g1_v, g2_v)[s]
                for i in range(2 * B // IDX_CHUNK):
                    sl = pl.ds(i * IDX_CHUNK, IDX_CHUNK)
                    pltpu.make_async_copy(
                        tab.at[iref.at[sl]], gref.at[sl], sem_g).wait()

        def drain_out(ci, buf):
            out_v, sem_o = buf[10], buf[12]
            blk0 = chunk_base(ci) // 128
            pltpu.make_async_copy(
                out_v, out_hbm.at[:, pl.ds(blk0, BBLK)], sem_o).wait()

        def shuffle_and_out(ci, buf):
            (uv_v, i0_v, i1_v, i2_v, bo0_v, bo1_v, bo2_v,
             g0_v, g1_v, g2_v, out_v, sem_g, sem_o) = buf
            blk0 = chunk_base(ci) // 128

            @plsc.parallel_loop(0, NG, 1, unroll=2)
            def shuf_body(g):
                q = iota + g * LANES
                qb = q >> 7
                ql = q & 127
                q3 = (q << 1) + q
                x7 = plsc.load_gather(bo0_v, [q])
                bo1 = plsc.load_gather(bo1_v, [q])
                bo2 = plsc.load_gather(bo2_v, [q])
                for s, col, j in _COLMAP:
                    if s == 0:
                        val = plsc.load_gather(g0_v, [q3 + col, x7])
                    else:
                        w = (bo1, bo2)[s - 1] + col
                        val = plsc.load_gather(
                            (g1_v, g2_v)[s - 1], [w >> 4, w & 15])
                    plsc.store_scatter(
                        out_v,
                        [zeros + (j % 9), qb, zeros + (j // 9), ql], val)

            pltpu.async_copy(out_v, out_hbm.at[:, pl.ds(blk0, BBLK)], sem_o)

        fire(0, bufA)

        def pair_body(cp, _):
            ca = cp * 2
            cb = ca + 1
            fire(cb, bufB)
            drain_gathers(bufA)

            @pl.when(cp > 0)
            def _older_a():
                drain_out(ca, bufA)

            shuffle_and_out(ca, bufA)

            @pl.when(cp < n_pairs - 1)
            def _next_a():
                fire(ca + 2, bufA)

            drain_gathers(bufB)

            @pl.when(cp > 0)
            def _older_b():
                drain_out(cb, bufB)

            shuffle_and_out(cb, bufB)
            return _

        lax.fori_loop(0, n_pairs, pair_body, None)
        drain_out(0, bufA)
        drain_out(0, bufB)

    out = sc_kernel(uv_blk, t0_rows, *tabs)
    return (out[:, :, :3, :]
            .transpose(1, 3, 2, 0)
            .reshape(N, 3, 9))


# speedup vs baseline: 20.8252x; 1.1044x over previous
import functools

import jax
import jax.numpy as jnp
from jax import lax
from jax.experimental import pallas as pl
from jax.experimental.pallas import tpu as pltpu
from jax.experimental.pallas import tpu_sc as plsc

N = 1048576
NBLK = N // 128
LANES = 16
B = 256
BBLK = B // 128
NG = B // LANES
IDX_CHUNK = 128

PLANE_ROWS = 2048 * 2048 // 8

_COLMAP = []
for _c in range(3):
    _COLMAP.append((0, _c, _c * 9 + 0))
    for _k in range(3):
        _COLMAP.append((1, 3 * _c + _k, _c * 9 + 1 + _k))
    for _k in range(5):
        _COLMAP.append((2, 5 * _c + _k, _c * 9 + 4 + _k))


def _scratch_set():
    return [
        pltpu.VMEM((BBLK, 2, 128), jnp.float32),
        pltpu.VMEM((3 * B,), jnp.int32),
        pltpu.VMEM((2 * B,), jnp.int32),
        pltpu.VMEM((2 * B,), jnp.int32),
        pltpu.VMEM((B,), jnp.int32),
        pltpu.VMEM((B,), jnp.int32),
        pltpu.VMEM((B,), jnp.int32),
        pltpu.VMEM((3 * B, 8), jnp.float32),
        pltpu.VMEM((2 * B, 16), jnp.float32),
        pltpu.VMEM((2 * B, 16), jnp.float32),
        pltpu.VMEM((9, BBLK, 4, 128), jnp.float32),
        pltpu.SemaphoreType.DMA,
        pltpu.SemaphoreType.DMA,
    ]


def kernel(uv_coords, tex0, tex1, tex2):
    t0_rows = (tex0.transpose(2, 0, 1)
               .reshape(3, 256, 8, 16, 128)
               .transpose(0, 1, 3, 2, 4)
               .reshape(3 * PLANE_ROWS, 8))
    tabs = [tex1.reshape(-1, 16), tex2.reshape(-1, 16)]
    maxrow = [t.shape[0] - 1 for t in tabs]
    uv_blk = uv_coords.reshape(NBLK, 128, 2).transpose(0, 2, 1)

    info = plsc.get_sparse_core_info()
    nc, ns = info.num_cores, info.num_subcores
    nw = nc * ns
    pts_per_tile = N // nw
    n_chunks = pts_per_tile // B
    n_pairs = n_chunks // 2

    @functools.partial(
        pl.kernel,
        out_type=jax.ShapeDtypeStruct((9, NBLK, 4, 128), jnp.float32),
        mesh=plsc.VectorSubcoreMesh(core_axis_name="c", subcore_axis_name="s"),
        compiler_params=pltpu.CompilerParams(
            needs_layout_passes=False, use_tc_tiling_on_sc=False),
        scratch_types=_scratch_set() + _scratch_set(),
    )
    def sc_kernel(uv_hbm, t0_hbm, t1_hbm, t2_hbm, out_hbm, *scr):
        bufA, bufB = scr[:13], scr[13:]
        wid = lax.axis_index("s") * nc + lax.axis_index("c")
        iota = lax.iota(jnp.int32, LANES)
        zeros = jnp.zeros((LANES,), jnp.int32)

        def chunk_base(ci):
            return wid * pts_per_tile + ci * B

        def fire(ci, buf):
            (uv_v, i0_v, i1_v, i2_v, bo0_v, bo1_v, bo2_v,
             g0_v, g1_v, g2_v, out_v, sem_g, sem_o) = buf
            blk0 = chunk_base(ci) // 128
            pltpu.sync_copy(uv_hbm.at[pl.ds(blk0, BBLK)], uv_v)

            @plsc.parallel_loop(0, NG, 1, unroll=2)
            def gen_body(g):
                q = iota + g * LANES
                qb = q >> 7
                ql = q & 127
                q2 = q << 1
                q3 = q2 + q
                u = plsc.load_gather(uv_v, [qb, zeros, ql])
                v = plsc.load_gather(uv_v, [qb, zeros + 1, ql])
                ix = jnp.clip((u * 2048.0).astype(jnp.int32), 0, 2047)
                iy = jnp.clip((v * 2048.0).astype(jnp.int32), 0, 2047)
                w = ((((iy >> 3) << 4) + (ix >> 7)) << 10) \
                    + ((iy & 7) << 7) + (ix & 127)
                r0 = w >> 3
                plsc.store_scatter(i0_v, [q3], r0)
                plsc.store_scatter(i0_v, [q3 + 1], r0 + PLANE_ROWS)
                plsc.store_scatter(i0_v, [q3 + 2], r0 + 2 * PLANE_ROWS)
                plsc.store_scatter(bo0_v, [q], ix & 7)
                for s, d in ((0, 9), (1, 15)):
                    sh = s + 1
                    texel = ((iy >> sh) << (11 - sh)) + (ix >> sh)
                    o = texel * d
                    r = o >> 4
                    r2 = jnp.minimum(r + 1, maxrow[s])
                    iref = (i1_v, i2_v)[s]
                    plsc.store_scatter(iref, [q2], r)
                    plsc.store_scatter(iref, [q2 + 1], r2)
                    plsc.store_scatter((bo1_v, bo2_v)[s], [q],
                                       (q << 5) + (o & 15))

            for i in range(3 * B // IDX_CHUNK):
                sl = pl.ds(i * IDX_CHUNK, IDX_CHUNK)
                pltpu.async_copy(t0_hbm.at[i0_v.at[sl]], g0_v.at[sl], sem_g)
            for s in range(2):
                tab = (t1_hbm, t2_hbm)[s]
                iref = (i1_v, i2_v)[s]
                gref = (g1_v, g2_v)[s]
                for i in range(2 * B // IDX_CHUNK):
                    sl = pl.ds(i * IDX_CHUNK, IDX_CHUNK)
                    pltpu.async_copy(tab.at[iref.at[sl]], gref.at[sl], sem_g)

        def drain_gathers(buf):
            (uv_v, i0_v, i1_v, i2_v, bo0_v, bo1_v, bo2_v,
             g0_v, g1_v, g2_v, out_v, sem_g, sem_o) = buf
            for i in range(3 * B // IDX_CHUNK):
                sl = pl.ds(i * IDX_CHUNK, IDX_CHUNK)
                pltpu.make_async_copy(
                    t0_hbm.at[i0_v.at[sl]], g0_v.at[sl], sem_g).wait()
            for s in range(2):
                tab = (t1_hbm, t2_hbm)[s]
                iref = (i1_v, i2_v)[s]
                gref = (g1_v, g2_v)[s]
                for i in range(2 * B // IDX_CHUNK):
                    sl = pl.ds(i * IDX_CHUNK, IDX_CHUNK)
                    pltpu.make_async_copy(
                        tab.at[iref.at[sl]], gref.at[sl], sem_g).wait()

        def drain_out(ci, buf):
            out_v, sem_o = buf[10], buf[12]
            blk0 = chunk_base(ci) // 128
            pltpu.make_async_copy(
                out_v, out_hbm.at[:, pl.ds(blk0, BBLK)], sem_o).wait()

        def shuffle_and_out(ci, buf):
            (uv_v, i0_v, i1_v, i2_v, bo0_v, bo1_v, bo2_v,
             g0_v, g1_v, g2_v, out_v, sem_g, sem_o) = buf
            blk0 = chunk_base(ci) // 128

            @plsc.parallel_loop(0, NG, 1, unroll=2)
            def shuf_body(g):
                q = iota + g * LANES
                qb = q >> 7
                ql = q & 127
                q3 = (q << 1) + q
                x7 = plsc.load_gather(bo0_v, [q])
                bo1 = plsc.load_gather(bo1_v, [q])
                bo2 = plsc.load_gather(bo2_v, [q])
                for s, col, j in _COLMAP:
                    if s == 0:
                        val = plsc.load_gather(g0_v, [q3 + col, x7])
                    else:
                        w = (bo1, bo2)[s - 1] + col
                        val = plsc.load_gather(
                            (g1_v, g2_v)[s - 1], [w >> 4, w & 15])
                    plsc.store_scatter(
                        out_v,
                        [zeros + (j % 9), qb, zeros + (j // 9), ql], val)

            pltpu.async_copy(out_v, out_hbm.at[:, pl.ds(blk0, BBLK)], sem_o)

        fire(0, bufA)

        def pair_body(cp, _):
            ca = cp * 2
            cb = ca + 1
            fire(cb, bufB)
            drain_gathers(bufA)

            @pl.when(cp > 0)
            def _older_a():
                drain_out(ca, bufA)

            shuffle_and_out(ca, bufA)

            @pl.when(cp < n_pairs - 1)
            def _next_a():
                fire(ca + 2, bufA)

            drain_gathers(bufB)

            @pl.when(cp > 0)
            def _older_b():
                drain_out(cb, bufB)

            shuffle_and_out(cb, bufB)
            return _

        lax.fori_loop(0, n_pairs, pair_body, None)
        drain_out(0, bufA)
        drain_out(0, bufB)

    out = sc_kernel(uv_blk, t0_rows, *tabs)
    return (out.transpose(1, 3, 2, 0)
            .reshape(N, 4, 9)[:, :3, :])
